# baseline (speedup 1.0000x reference)
import functools

import jax
import jax.numpy as jnp
from jax import lax
from jax.experimental import pallas as pl
from jax.experimental.pallas import tpu as pltpu
from jax.experimental.pallas import tpu_sc as plsc

_V = 100000
_D = 64
_H = 128
_L = 200

_BPW = 8
_NACT = _L // _BPW
_NW = 32

_BV = 12800
_NB = (_V + _BV - 1) // _BV


def _sc_gather_sum(inputs_i32, emb):
    mesh = plsc.VectorSubcoreMesh(core_axis_name="c", subcore_axis_name="s")

    @functools.partial(
        pl.kernel,
        mesh=mesh,
        out_type=jax.ShapeDtypeStruct((_NW, _D), jnp.float32),
        compiler_params=pltpu.CompilerParams(use_tc_tiling_on_sc=False),
        scratch_types=[
            pltpu.VMEM((_BPW,), jnp.int32),
            pltpu.VMEM((_BPW, _D), jnp.float32),
            pltpu.VMEM((1, _D), jnp.float32),
            pltpu.SemaphoreType.DMA,
        ],
    )
    def gather_sum(idx_hbm, table_hbm, out_hbm, idx_v, rows_v, sum_v, sem):
        wid = lax.axis_index("s") * 2 + lax.axis_index("c")

        @pl.when(wid < _NACT)
        def _active():
            pltpu.sync_copy(idx_hbm.at[pl.ds(wid * _BPW, _BPW)], idx_v)
            pltpu.async_copy(table_hbm.at[idx_v], rows_v, sem).wait()
            for d in range(_D // 16):
                acc = rows_v[0, pl.ds(d * 16, 16)]
                for r in range(1, _BPW):
                    acc = acc + rows_v[r, pl.ds(d * 16, 16)]
                sum_v[0, pl.ds(d * 16, 16)] = acc

        @pl.when(wid >= _NACT)
        def _idle():
            for d in range(_D // 16):
                sum_v[0, pl.ds(d * 16, 16)] = jnp.zeros((16,), jnp.float32)

        pltpu.sync_copy(sum_v, out_hbm.at[pl.ds(wid, 1)])

    return gather_sum(inputs_i32, emb)


def _mlp_body(parts_ref, w1_ref, b1_ref, w2a_ref, w2b_ref, b2_ref, out_ref,
              h_ref, m_ref, s_ref):
    j = pl.program_id(0)

    @pl.when(j == 0)
    def _init():
        embeds = jnp.sum(parts_ref[...], axis=0, keepdims=True)
        h = lax.dot_general(embeds, w1_ref[...], (((1,), (0,)), ((), ())),
                            preferred_element_type=jnp.float32)
        h_ref[...] = jnp.maximum(h + b1_ref[...], 0.0)
        m_ref[...] = jnp.full((1, 1), -jnp.inf, jnp.float32)
        s_ref[...] = jnp.zeros((1, 1), jnp.float32)

    za = lax.dot_general(h_ref[:, :_H // 2], w2a_ref[...],
                         (((1,), (0,)), ((), ())),
                         preferred_element_type=jnp.float32)
    zb = lax.dot_general(h_ref[:, _H // 2:], w2b_ref[...],
                         (((1,), (0,)), ((), ())),
                         preferred_element_type=jnp.float32)
    z = za + zb + b2_ref[...]
    out_ref[pl.ds(j, 1), :] = z

    col = j * _BV + lax.broadcasted_iota(jnp.int32, (1, _BV), 1)
    zm = jnp.where(col < _V, z, -jnp.inf)

    m_old = m_ref[...]
    m_new = jnp.maximum(m_old, jnp.max(zm, axis=1, keepdims=True))
    s_ref[...] = (s_ref[...] * jnp.exp(m_old - m_new)
                  + jnp.sum(jnp.exp(zm - m_new), axis=1, keepdims=True))
    m_ref[...] = m_new

    @pl.when(j == pl.num_programs(0) - 1)
    def _finalize():
        lse = m_ref[...] + jnp.log(s_ref[...])
        out_ref[...] = out_ref[...] - lse


def _tc_mlp_logsoftmax(partials, W1, b1, W2, b2):
    return pl.pallas_call(
        _mlp_body,
        grid=(_NB,),
        in_specs=[
            pl.BlockSpec((_NW, _D), lambda j: (0, 0)),
            pl.BlockSpec((_D, _H), lambda j: (0, 0)),
            pl.BlockSpec((1, _H), lambda j: (0, 0)),
            pl.BlockSpec((_H // 2, _BV), lambda j: (0, j)),
            pl.BlockSpec((_H // 2, _BV), lambda j: (1, j)),
            pl.BlockSpec((1, _BV), lambda j: (0, j)),
        ],
        out_specs=pl.BlockSpec((_NB, _BV), lambda j: (0, 0)),
        out_shape=jax.ShapeDtypeStruct((_NB, _BV), jnp.float32),
        scratch_shapes=[
            pltpu.VMEM((1, _H), jnp.float32),
            pltpu.VMEM((1, 1), jnp.float32),
            pltpu.VMEM((1, 1), jnp.float32),
        ],
    )(partials, W1, b1.reshape(1, _H), W2, W2, b2.reshape(1, _V))


def kernel(inputs, emb, W1, b1, W2, b2):
    flat = W2.reshape(-1)
    CH = 400000
    PIECE = 50000
    NP = CH // PIECE
    mesh = plsc.VectorSubcoreMesh(core_axis_name="c", subcore_axis_name="s")

    @functools.partial(
        pl.kernel,
        mesh=mesh,
        out_type=jax.ShapeDtypeStruct((_NW, 16), jnp.float32),
        compiler_params=pltpu.CompilerParams(use_tc_tiling_on_sc=False),
        scratch_types=[
            pltpu.VMEM((PIECE,), jnp.float32),
            pltpu.VMEM((PIECE,), jnp.float32),
            pltpu.VMEM((1, 16), jnp.float32),
            pltpu.SemaphoreType.DMA,
            pltpu.SemaphoreType.DMA,
        ],
    )
    def probe(flat_hbm, out_hbm, buf0, buf1, res_v, sem0, sem1):
        wid = lax.axis_index("s") * 2 + lax.axis_index("c")
        base = wid * CH
        bufs, sems = (buf0, buf1), (sem0, sem1)
        copies = []
        for i in range(NP):
            if i >= 2:
                copies[i - 2].wait()
            copies.append(pltpu.async_copy(
                flat_hbm.at[pl.ds(base + i * PIECE, PIECE)],
                bufs[i % 2], sems[i % 2]))
        copies[-2].wait()
        copies[-1].wait()
        res_v[0, pl.ds(0, 16)] = buf0[pl.ds(0, 16)] + buf1[pl.ds(0, 16)]
        pltpu.sync_copy(res_v, out_hbm.at[pl.ds(wid, 1)])

    return probe(flat)


# speedup vs baseline: 1.0159x; 1.0159x over previous
import functools

import jax
import jax.numpy as jnp
from jax import lax
from jax.experimental import pallas as pl
from jax.experimental.pallas import tpu as pltpu
from jax.experimental.pallas import tpu_sc as plsc

_V = 100000
_D = 64
_H = 128
_L = 200

_BPW = 8
_NACT = _L // _BPW
_NW = 32

_BV = 12800
_NB = (_V + _BV - 1) // _BV


def _sc_gather_sum(inputs_i32, emb):
    mesh = plsc.VectorSubcoreMesh(core_axis_name="c", subcore_axis_name="s")

    @functools.partial(
        pl.kernel,
        mesh=mesh,
        out_type=jax.ShapeDtypeStruct((_NW, _D), jnp.float32),
        compiler_params=pltpu.CompilerParams(use_tc_tiling_on_sc=False),
        scratch_types=[
            pltpu.VMEM((_BPW,), jnp.int32),
            pltpu.VMEM((_BPW, _D), jnp.float32),
            pltpu.VMEM((1, _D), jnp.float32),
            pltpu.SemaphoreType.DMA,
        ],
    )
    def gather_sum(idx_hbm, table_hbm, out_hbm, idx_v, rows_v, sum_v, sem):
        wid = lax.axis_index("s") * 2 + lax.axis_index("c")

        @pl.when(wid < _NACT)
        def _active():
            pltpu.sync_copy(idx_hbm.at[pl.ds(wid * _BPW, _BPW)], idx_v)
            pltpu.async_copy(table_hbm.at[idx_v], rows_v, sem).wait()
            for d in range(_D // 16):
                acc = rows_v[0, pl.ds(d * 16, 16)]
                for r in range(1, _BPW):
                    acc = acc + rows_v[r, pl.ds(d * 16, 16)]
                sum_v[0, pl.ds(d * 16, 16)] = acc

        @pl.when(wid >= _NACT)
        def _idle():
            for d in range(_D // 16):
                sum_v[0, pl.ds(d * 16, 16)] = jnp.zeros((16,), jnp.float32)

        pltpu.sync_copy(sum_v, out_hbm.at[pl.ds(wid, 1)])

    return gather_sum(inputs_i32, emb)


def _mlp_body(parts_ref, w1_ref, b1_ref, w2a_ref, w2b_ref, b2_ref, out_ref,
              h_ref, m_ref, s_ref):
    j = pl.program_id(0)

    @pl.when(j == 0)
    def _init():
        embeds = jnp.sum(parts_ref[...], axis=0, keepdims=True)
        h = lax.dot_general(embeds, w1_ref[...], (((1,), (0,)), ((), ())),
                            preferred_element_type=jnp.float32)
        h_ref[...] = jnp.maximum(h + b1_ref[...], 0.0)
        m_ref[...] = jnp.full((1, 1), -jnp.inf, jnp.float32)
        s_ref[...] = jnp.zeros((1, 1), jnp.float32)

    za = lax.dot_general(h_ref[:, :_H // 2], w2a_ref[...],
                         (((1,), (0,)), ((), ())),
                         preferred_element_type=jnp.float32)
    zb = lax.dot_general(h_ref[:, _H // 2:], w2b_ref[...],
                         (((1,), (0,)), ((), ())),
                         preferred_element_type=jnp.float32)
    z = za + zb + b2_ref[...]
    out_ref[pl.ds(j, 1), :] = z

    col = j * _BV + lax.broadcasted_iota(jnp.int32, (1, _BV), 1)
    zm = jnp.where(col < _V, z, -jnp.inf)

    m_old = m_ref[...]
    m_new = jnp.maximum(m_old, jnp.max(zm, axis=1, keepdims=True))
    s_ref[...] = (s_ref[...] * jnp.exp(m_old - m_new)
                  + jnp.sum(jnp.exp(zm - m_new), axis=1, keepdims=True))
    m_ref[...] = m_new

    @pl.when(j == pl.num_programs(0) - 1)
    def _finalize():
        lse = m_ref[...] + jnp.log(s_ref[...])
        out_ref[...] = out_ref[...] - lse


def _tc_mlp_logsoftmax(partials, W1, b1, W2, b2):
    return pl.pallas_call(
        _mlp_body,
        grid=(_NB,),
        in_specs=[
            pl.BlockSpec((_NW, _D), lambda j: (0, 0)),
            pl.BlockSpec((_D, _H), lambda j: (0, 0)),
            pl.BlockSpec((1, _H), lambda j: (0, 0)),
            pl.BlockSpec((_H // 2, _BV), lambda j: (0, j)),
            pl.BlockSpec((_H // 2, _BV), lambda j: (1, j)),
            pl.BlockSpec((1, _BV), lambda j: (0, j)),
        ],
        out_specs=pl.BlockSpec((_NB, _BV), lambda j: (0, 0)),
        out_shape=jax.ShapeDtypeStruct((_NB, _BV), jnp.float32),
        scratch_shapes=[
            pltpu.VMEM((1, _H), jnp.float32),
            pltpu.VMEM((1, 1), jnp.float32),
            pltpu.VMEM((1, 1), jnp.float32),
        ],
    )(partials, W1, b1.reshape(1, _H), W2, W2, b2.reshape(1, _V))


def kernel(inputs, emb, W1, b1, W2, b2):
    partials = _sc_gather_sum(inputs.astype(jnp.int32), emb)
    out = _tc_mlp_logsoftmax(partials, W1, b1, W2, b2)
    return out.reshape(1, _NB * _BV)[:, :_V]


# speedup vs baseline: 1.3976x; 1.3757x over previous
import jax
import jax.numpy as jnp
from jax import lax
from jax.experimental import pallas as pl
from jax.experimental.pallas import tpu as pltpu

_V = 100000
_D = 64
_H = 128
_L = 200

_BV = 12800
_NB = (_V + _BV - 1) // _BV


def _mlp_body(idx_ref, emb_ref, w1_ref, b1_ref, w2a_ref, w2b_ref, b2_ref,
              out_ref, rows_ref, h_ref, m_ref, s_ref, sem):
    j = pl.program_id(0)

    @pl.when(j == 0)
    def _init():
        copies = [
            pltpu.make_async_copy(
                emb_ref.at[pl.ds(idx_ref[t], 1)],
                rows_ref.at[pl.ds(t, 1)], sem)
            for t in range(_L)
        ]
        for c in copies:
            c.start()
        for c in copies:
            c.wait()
        embeds = jnp.sum(rows_ref[...], axis=0, keepdims=True)
        h = lax.dot_general(embeds, w1_ref[...], (((1,), (0,)), ((), ())),
                            preferred_element_type=jnp.float32)
        h_ref[...] = jnp.maximum(h + b1_ref[...], 0.0)
        m_ref[...] = jnp.full((1, 1), -jnp.inf, jnp.float32)
        s_ref[...] = jnp.zeros((1, 1), jnp.float32)

    za = lax.dot_general(h_ref[:, :_H // 2], w2a_ref[...],
                         (((1,), (0,)), ((), ())),
                         preferred_element_type=jnp.float32)
    zb = lax.dot_general(h_ref[:, _H // 2:], w2b_ref[...],
                         (((1,), (0,)), ((), ())),
                         preferred_element_type=jnp.float32)
    z = za + zb + b2_ref[...]
    out_ref[pl.ds(j, 1), :] = z

    col = j * _BV + lax.broadcasted_iota(jnp.int32, (1, _BV), 1)
    zm = jnp.where(col < _V, z, -jnp.inf)

    m_old = m_ref[...]
    m_new = jnp.maximum(m_old, jnp.max(zm, axis=1, keepdims=True))
    s_ref[...] = (s_ref[...] * jnp.exp(m_old - m_new)
                  + jnp.sum(jnp.exp(zm - m_new), axis=1, keepdims=True))
    m_ref[...] = m_new

    @pl.when(j == pl.num_programs(0) - 1)
    def _finalize():
        lse = m_ref[...] + jnp.log(s_ref[...])
        out_ref[...] = out_ref[...] - lse


def kernel(inputs, emb, W1, b1, W2, b2):
    out = pl.pallas_call(
        _mlp_body,
        grid=(_NB,),
        in_specs=[
            pl.BlockSpec(memory_space=pltpu.MemorySpace.SMEM),
            pl.BlockSpec(memory_space=pltpu.MemorySpace.HBM),
            pl.BlockSpec((_D, _H), lambda j: (0, 0)),
            pl.BlockSpec((1, _H), lambda j: (0, 0)),
            pl.BlockSpec((_H // 2, _BV), lambda j: (0, j)),
            pl.BlockSpec((_H // 2, _BV), lambda j: (1, j)),
            pl.BlockSpec((1, _BV), lambda j: (0, j)),
        ],
        out_specs=pl.BlockSpec((_NB, _BV), lambda j: (0, 0)),
        out_shape=jax.ShapeDtypeStruct((_NB, _BV), jnp.float32),
        scratch_shapes=[
            pltpu.VMEM((_L, _D), jnp.float32),
            pltpu.VMEM((1, _H), jnp.float32),
            pltpu.VMEM((1, 1), jnp.float32),
            pltpu.VMEM((1, 1), jnp.float32),
            pltpu.SemaphoreType.DMA,
        ],
    )(inputs.astype(jnp.int32), emb, W1, b1.reshape(1, _H), W2, W2,
      b2.reshape(1, _V))
    return out.reshape(1, _NB * _BV)[:, :_V]


# speedup vs baseline: 1.4294x; 1.0227x over previous
import jax
import jax.numpy as jnp
from jax import lax
from jax.experimental import pallas as pl
from jax.experimental.pallas import tpu as pltpu

_V = 100000
_D = 64
_H = 128
_L = 200

_KC = 16
_NC = _H // _KC
_R = 5


def _body(idx_ref, emb_ref, w2_ref, w1_ref, b1_ref, b2_ref, out_ref,
          rows_v, acc_v, w2buf, gsem, bsem, osem):
    def chunk_copy(c):
        return pltpu.make_async_copy(
            w2_ref.at[pl.ds(c * _KC, _KC), :],
            w2buf.at[c % _R], bsem.at[c % _R])

    for c in range(_R):
        chunk_copy(c).start()

    for t in range(_L):
        pltpu.make_async_copy(
            emb_ref.at[pl.ds(idx_ref[t], 1)],
            rows_v.at[pl.ds(t, 1)], gsem).start()
    pltpu.make_async_copy(emb_ref.at[pl.ds(0, _L)], rows_v, gsem).wait()

    embeds = jnp.sum(rows_v[...], axis=0, keepdims=True)
    h = lax.dot_general(embeds, w1_ref[...], (((1,), (0,)), ((), ())),
                        preferred_element_type=jnp.float32)
    h = jnp.maximum(h + b1_ref[...], 0.0)

    for c in range(_NC):
        r = c % _R
        chunk_copy(c).wait()
        zc = lax.dot_general(h[:, c * _KC:(c + 1) * _KC], w2buf[r],
                             (((1,), (0,)), ((), ())),
                             preferred_element_type=jnp.float32)
        if c == 0:
            acc_v[...] = zc + b2_ref[...]
        else:
            acc_v[...] = acc_v[...] + zc
        if c + _R < _NC:
            chunk_copy(c + _R).start()

    z = acc_v[...]
    m = jnp.max(z, axis=1, keepdims=True)
    s = jnp.sum(jnp.exp(z - m), axis=1, keepdims=True)
    acc_v[...] = z - (m + jnp.log(s))
    cp = pltpu.make_async_copy(acc_v, out_ref, osem)
    cp.start()
    cp.wait()


def kernel(inputs, emb, W1, b1, W2, b2):
    return pl.pallas_call(
        _body,
        in_specs=[
            pl.BlockSpec(memory_space=pltpu.MemorySpace.SMEM),
            pl.BlockSpec(memory_space=pltpu.MemorySpace.HBM),
            pl.BlockSpec(memory_space=pltpu.MemorySpace.HBM),
            pl.BlockSpec((_D, _H), lambda: (0, 0)),
            pl.BlockSpec((1, _H), lambda: (0, 0)),
            pl.BlockSpec((1, _V), lambda: (0, 0)),
        ],
        out_specs=pl.BlockSpec(memory_space=pltpu.MemorySpace.HBM),
        out_shape=jax.ShapeDtypeStruct((1, _V), jnp.float32),
        scratch_shapes=[
            pltpu.VMEM((_L, _D), jnp.float32),
            pltpu.VMEM((1, _V), jnp.float32),
            pltpu.VMEM((_R, _KC, _V), jnp.float32),
            pltpu.SemaphoreType.DMA,
            pltpu.SemaphoreType.DMA((_R,)),
            pltpu.SemaphoreType.DMA,
        ],
    )(inputs.astype(jnp.int32), emb, W2, W1, b1.reshape(1, _H),
      b2.reshape(1, _V))
